# baseline (speedup 1.0000x reference)
import jax
import jax.numpy as jnp
from jax import lax
from jax.experimental import pallas as pl
from jax.experimental.pallas import tpu as pltpu
from jax.experimental.pallas import tpu_sc as plsc

DIM = 32
BATCH = 16384

NC = 2
NS = 16
NW = NC * NS
BPW = BATCH // NW
CHUNK = 128
NCHUNK = BPW // CHUNK
GROUPS = BPW // 16


def _mure_body(u_idx_hbm, r_idx_hbm, v_idx_hbm, E_hbm, Wu_hbm, rv_hbm,
               bs_hbm, bo_hbm, out_hbm,
               idx_u, idx_r, idx_v, u_rows, v_rows, ru_rows, rv_rows,
               bs_v, bo_v, out_v, sem):
    wid = lax.axis_index("s") * NC + lax.axis_index("c")
    base = wid * BPW

    for j in range(NCHUNK):
        sl = pl.ds(base + j * CHUNK, CHUNK)
        pltpu.sync_copy(u_idx_hbm.at[sl], idx_u.at[j])
        pltpu.sync_copy(r_idx_hbm.at[sl], idx_r.at[j])
        pltpu.sync_copy(v_idx_hbm.at[sl], idx_v.at[j])

    cps = []
    for j in range(NCHUNK):
        sl = pl.ds(j * CHUNK, CHUNK)
        cps.append(pltpu.async_copy(E_hbm.at[idx_u.at[j]], u_rows.at[sl], sem))
        cps.append(pltpu.async_copy(E_hbm.at[idx_v.at[j]], v_rows.at[sl], sem))
        cps.append(pltpu.async_copy(Wu_hbm.at[idx_r.at[j]], ru_rows.at[sl], sem))
        cps.append(pltpu.async_copy(rv_hbm.at[idx_r.at[j]], rv_rows.at[sl], sem))
        cps.append(pltpu.async_copy(bs_hbm.at[idx_u.at[j]], bs_v.at[sl], sem))
        cps.append(pltpu.async_copy(bo_hbm.at[idx_v.at[j]], bo_v.at[sl], sem))
    for cp in cps:
        cp.wait()

    lane = lax.iota(jnp.int32, 16)

    def group(g, carry):
        rows16 = g * 16 + lane
        acc = jnp.zeros((16,), jnp.float32)
        for d in range(DIM):
            dd = jnp.full((16,), d, jnp.int32)
            uu = plsc.load_gather(u_rows, [rows16, dd])
            ru = plsc.load_gather(ru_rows, [rows16, dd])
            vv = plsc.load_gather(v_rows, [rows16, dd])
            rr = plsc.load_gather(rv_rows, [rows16, dd])
            t = uu * ru - vv - rr
            acc = acc + t * t
        sl = pl.ds(g * 16, 16)
        out_v[sl] = bs_v[sl] + bo_v[sl] - acc
        return carry

    lax.fori_loop(0, GROUPS, group, 0)
    pltpu.sync_copy(out_v, out_hbm.at[pl.ds(base, BPW)])


@jax.jit
def _mure_sc(u_idx, r_idx, v_idx, E, Wu, rv, bs, bo):
    mesh = plsc.VectorSubcoreMesh(core_axis_name="c", subcore_axis_name="s")
    return pl.kernel(
        _mure_body,
        mesh=mesh,
        compiler_params=pltpu.CompilerParams(
            needs_layout_passes=False, use_tc_tiling_on_sc=False),
        out_type=jax.ShapeDtypeStruct((BATCH,), jnp.float32),
        scratch_types=[
            pltpu.VMEM((NCHUNK, CHUNK), jnp.int32),
            pltpu.VMEM((NCHUNK, CHUNK), jnp.int32),
            pltpu.VMEM((NCHUNK, CHUNK), jnp.int32),
            pltpu.VMEM((BPW, DIM), jnp.float32),
            pltpu.VMEM((BPW, DIM), jnp.float32),
            pltpu.VMEM((BPW, DIM), jnp.float32),
            pltpu.VMEM((BPW, DIM), jnp.float32),
            pltpu.VMEM((BPW,), jnp.float32),
            pltpu.VMEM((BPW,), jnp.float32),
            pltpu.VMEM((BPW,), jnp.float32),
            pltpu.SemaphoreType.DMA,
        ],
    )(u_idx, r_idx, v_idx, E, Wu, rv, bs, bo)


def kernel(u_idx, r_idx, v_idx, E, Wu, rv, bs, bo):
    return _mure_sc(u_idx, r_idx, v_idx, E, Wu, rv, bs, bo)
